# baseline (speedup 1.0000x reference)
import jax
import jax.numpy as jnp
from jax import lax
from jax.experimental import pallas as pl
from jax.experimental.pallas import tpu as pltpu
from jax.experimental.pallas import tpu_sc as plsc

NC = 2
NS = 16
L = 16

B, N, E, K = 4, 25000, 800000, 64
BPC = B // NC
TPB = NS // BPC
NLOC = 25088
Q = NLOC // TPB
NPAD2 = BPC * NLOC
CH = 8192
EPT = 13 * CH
E_PAD = EPT * TPB

MAGIC = 0x5F3759DF


def _rsqrt16(v):
    ib = lax.bitcast_convert_type(v, jnp.int32)
    y = lax.bitcast_convert_type(MAGIC - (ib >> 1), jnp.float32)
    half = v * 0.5
    for _ in range(3):
        y = y * (1.5 - half * y * y)
    return jnp.where(v > 0.5, y, 0.0)


def _bf16r(v):
    ib = lax.bitcast_convert_type(v, jnp.int32)
    r = ib + 0x7FFF + ((ib >> 16) & 1)
    r = jnp.bitwise_and(r, jnp.int32(-65536))
    return lax.bitcast_convert_type(r, jnp.float32)


def _body(ei, xp, dfi, wg, bg, wp, bp, wv, bv, wt, tm,
          pol_out, val_out,
          d_s, p_s, s_s, stage_s,
          s_priv, ploc,
          rawc, rawr,
          xv, dvv, pvv,
          cb0, cb1, cb2, cb3,
          dbuf, pbuf, vbuf, shbuf):
    cid = lax.axis_index("c")
    tid = lax.axis_index("s")
    bl = tid // TPB
    b = cid * BPC + bl
    lane8 = tid % TPB
    iot = lax.iota(jnp.int32, L)
    ones16 = jnp.ones((L,), jnp.float32)
    row_base = (b * 2 + 0) * E_PAD
    col_base = (b * 2 + 1) * E_PAD
    start = lane8 * EPT
    nbase = tid * Q

    def zero_priv():
        def z16(i, _):
            s_priv[pl.ds(i * L, L)] = jnp.zeros((L,), jnp.float32)
            return 0
        lax.fori_loop(0, NLOC // L, z16, 0)

    def stage_and_reduce(acc):
        pltpu.sync_copy(s_priv, stage_s.at[pl.ds(tid * NLOC, NLOC)])
        plsc.subcore_barrier()
        rbase = bl * TPB * NLOC + lane8 * Q
        pltpu.sync_copy(stage_s.at[pl.ds(rbase, Q)], acc)
        for r in range(1, TPB):
            pltpu.sync_copy(stage_s.at[pl.ds(rbase + r * NLOC, Q)], xv)

            def radd(i, _):
                acc[pl.ds(i * L, L)] = acc[pl.ds(i * L, L)] + xv[pl.ds(i * L, L)]
                return 0
            lax.fori_loop(0, Q // L, radd, 0)

    zero_priv()

    def deg_chunk(k, _):
        off = start + k * CH
        pltpu.sync_copy(ei.at[pl.ds(col_base + off, CH)], rawc)

        def cadd(i, _):
            plsc.addupdate_scatter(s_priv, [rawc[pl.ds(i * L, L)]], ones16)
            return 0
        lax.fori_loop(0, CH // L, cadd, 0)
        return 0
    lax.fori_loop(0, EPT // CH, deg_chunk, 0)
    stage_and_reduce(dvv)

    @pl.when(jnp.logical_and(cid == 0, tid == 0))
    def _selfloop_deg():
        dvv[pl.ds(0, L)] = dvv[pl.ds(0, L)] + jnp.where(iot == 0, 1.0, 0.0)

    pltpu.sync_copy(xp.at[pl.ds(cid * NPAD2 + nbase, Q)], xv)

    def dp16(i, _):
        dd = _rsqrt16(dvv[pl.ds(i * L, L)])
        dvv[pl.ds(i * L, L)] = dd
        pvv[pl.ds(i * L, L)] = dd * xv[pl.ds(i * L, L)]
        return 0
    lax.fori_loop(0, Q // L, dp16, 0)
    pltpu.sync_copy(dvv, d_s.at[pl.ds(nbase, Q)])
    pltpu.sync_copy(pvv, p_s.at[pl.ds(nbase, Q)])
    vbuf[...] = jnp.where(iot == 0, pvv[pl.ds(0, L)], 0.0)
    zero_priv()
    plsc.subcore_barrier()

    pltpu.sync_copy(p_s.at[pl.ds(bl * NLOC, NLOC)], ploc)

    def msg_chunk(k, _):
        off = start + k * CH
        pltpu.sync_copy(ei.at[pl.ds(row_base + off, CH)], rawr)
        pltpu.sync_copy(ei.at[pl.ds(col_base + off, CH)], rawc)

        def gadd(i, _):
            pd16 = plsc.load_gather(ploc, [rawr[pl.ds(i * L, L)]])
            plsc.addupdate_scatter(s_priv, [rawc[pl.ds(i * L, L)]], pd16)
            return 0
        lax.fori_loop(0, CH // L, gadd, 0)
        return 0
    lax.fori_loop(0, EPT // CH, msg_chunk, 0)

    @pl.when(jnp.logical_and(cid == 0, tid == 0))
    def _selfloop_msg():
        s_priv[pl.ds(0, L)] = s_priv[pl.ds(0, L)] + vbuf[...]

    stage_and_reduce(pvv)
    pltpu.sync_copy(d_s.at[pl.ds(nbase, Q)], xv)

    def smul(i, _):
        pvv[pl.ds(i * L, L)] = pvv[pl.ds(i * L, L)] * xv[pl.ds(i * L, L)]
        return 0
    lax.fori_loop(0, Q // L, smul, 0)
    pltpu.sync_copy(pvv, s_s.at[pl.ds(nbase, Q)])
    plsc.subcore_barrier()

    def lanesum(v):
        for sh in (8, 4, 2, 1):
            shbuf[...] = v
            v = v + plsc.load_gather(shbuf, [jnp.bitwise_and(iot + sh, L - 1)])
        return v

    @pl.when(tid < 2)
    def _value_head():
        base = tid * NLOC
        acc0 = jnp.zeros((L,), jnp.float32)

        def vchunk(k, acc):
            pltpu.sync_copy(s_s.at[pl.ds(base + k * Q, Q)], xv)

            def vacc(i, a):
                return a + xv[pl.ds(i * L, L)]
            return lax.fori_loop(0, Q // L, vacc, acc)
        acc0 = lax.fori_loop(0, N // Q, vchunk, acc0)
        tail = N - (N // Q) * Q
        nfull = tail // L
        rem = tail - nfull * L
        pltpu.sync_copy(s_s.at[pl.ds(base + (N // Q) * Q, tail)],
                        xv.at[pl.ds(0, tail)])

        def vacc2(i, a):
            return a + xv[pl.ds(i * L, L)]
        acc0 = lax.fori_loop(0, nfull, vacc2, acc0)
        acc0 = acc0 + jnp.where(iot < rem, xv[pl.ds(nfull * L, L)], 0.0)
        pltpu.sync_copy(wv, cb0)
        pltpu.sync_copy(wg, cb1)
        pltpu.sync_copy(bg, cb2)
        pltpu.sync_copy(bv, cb3)
        means = lanesum(acc0) / jnp.float32(N)
        emb16 = _bf16r(means * cb1[...] + cb2[...])
        valv = lanesum(emb16 * _bf16r(cb0[...])) + lanesum(cb3[...])
        vbuf[...] = jnp.where(iot == 0, valv, 0.0)
        pltpu.sync_copy(vbuf, val_out.at[pl.ds((cid * BPC + tid) * L, L)])

    @pl.when(jnp.logical_and(tid >= 2, tid < 4))
    def _policy_head():
        pb = tid - 2
        gb = cid * BPC + pb
        pltpu.sync_copy(dfi.at[pl.ds(gb * K, K)], dbuf)
        pltpu.sync_copy(s_s.at[pl.ds(pb * NLOC, NLOC)], ploc)
        pltpu.sync_copy(wp, cb0)
        pltpu.sync_copy(wg, cb1)
        pltpu.sync_copy(bg, cb2)
        pltpu.sync_copy(bp, cb3)
        bp_s = lanesum(cb3[...])
        wgv, bgv, wpv = cb1[...], cb2[...], cb0[...]
        wgc = [lanesum(jnp.where(iot == c, wgv, 0.0)) for c in range(8)]
        bgc = [lanesum(jnp.where(iot == c, bgv, 0.0)) for c in range(8)]
        wpc = [_bf16r(lanesum(jnp.where(iot == c, wpv, 0.0))) for c in range(8)]
        for j in range(K // L):
            sg16 = plsc.load_gather(ploc, [dbuf[pl.ds(j * L, L)]])
            acc = bp_s
            for c in range(8):
                acc = acc + _bf16r(sg16 * wgc[c] + bgc[c]) * wpc[c]
            pbuf[pl.ds(j * L, L)] = acc
        pltpu.sync_copy(wt, cb1)
        pltpu.sync_copy(tm, cb2)
        wait_l = lanesum(_bf16r(cb1[...]) * _bf16r(cb0[...])) + bp_s
        term_l = lanesum(_bf16r(cb2[...]) * _bf16r(cb0[...])) + bp_s
        pbuf[pl.ds(K, L)] = (jnp.where(iot == 0, wait_l, 0.0)
                             + jnp.where(iot == 1, term_l, 0.0))
        pltpu.sync_copy(pbuf, pol_out.at[pl.ds(gb * (K + L), K + L)])


@jax.jit
def _run(ei_pad, xp, dfi, wg, bg, wp, bp, wv, bv, wt, tm):
    f32, i32 = jnp.float32, jnp.int32
    kfn = pl.kernel(
        _body,
        out_type=[
            jax.ShapeDtypeStruct((B * (K + L),), f32),
            jax.ShapeDtypeStruct((B * L,), f32),
        ],
        mesh=plsc.VectorSubcoreMesh(core_axis_name="c", subcore_axis_name="s"),
        compiler_params=pltpu.CompilerParams(needs_layout_passes=False),
        scratch_types=[
            pltpu.VMEM_SHARED((NPAD2,), f32),
            pltpu.VMEM_SHARED((NPAD2,), f32),
            pltpu.VMEM_SHARED((NPAD2,), f32),
            pltpu.VMEM_SHARED((NS * NLOC,), f32),
            pltpu.VMEM((NLOC,), f32),
            pltpu.VMEM((NLOC,), f32),
            pltpu.VMEM((CH,), i32),
            pltpu.VMEM((CH,), i32),
            pltpu.VMEM((Q,), f32),
            pltpu.VMEM((Q,), f32),
            pltpu.VMEM((Q,), f32),
            pltpu.VMEM((L,), f32),
            pltpu.VMEM((L,), f32),
            pltpu.VMEM((L,), f32),
            pltpu.VMEM((L,), f32),
            pltpu.VMEM((K,), i32),
            pltpu.VMEM((K + L,), f32),
            pltpu.VMEM((L,), f32),
            pltpu.VMEM((L,), f32),
        ],
    )
    return kfn(ei_pad, xp, dfi, wg, bg, wp, bp, wv, bv, wt, tm)


def kernel(x, edge_index, defense_indices, W_gcn, b_gcn, wait_emb, term_emb,
           W_pol, b_pol, W_val, b_val):
    def pad16(a):
        a = a.reshape(-1).astype(jnp.float32)
        return jnp.pad(a, (0, L - a.shape[0]))

    wg, bg = pad16(W_gcn), pad16(b_gcn)
    wp, bp = pad16(W_pol), pad16(b_pol)
    wv, bv = pad16(W_val), pad16(b_val)
    wt, tm = pad16(wait_emb), pad16(term_emb)

    xp = jnp.pad(x.reshape(B, N), ((0, 0), (0, NLOC - N))).reshape(-1)
    pad_blk = jnp.full((B, 2, E_PAD - E), N, dtype=jnp.int32)
    ei_pad = jnp.concatenate([edge_index, pad_blk], axis=2).reshape(-1)

    pol_pad, val_pad = _run(ei_pad, xp, defense_indices.reshape(-1), wg, bg,
                            wp, bp, wv, bv, wt, tm)
    return pol_pad.reshape(B, K + L)[:, : K + 2], val_pad.reshape(B, L)[:, 0]


# speedup vs baseline: 1.1595x; 1.1595x over previous
import jax
import jax.numpy as jnp
from jax import lax
from jax.experimental import pallas as pl
from jax.experimental.pallas import tpu as pltpu
from jax.experimental.pallas import tpu_sc as plsc

NC = 2
NS = 16
L = 16

B, N, E, K = 4, 25000, 800000, 64
BPC = B // NC
TPB = NS // BPC
NLOC = 25088
Q = NLOC // TPB
NPAD2 = BPC * NLOC
CH = 2048
EPT = 49 * CH
E_PAD = EPT * TPB

MAGIC = 0x5F3759DF


def _rsqrt16(v):
    ib = lax.bitcast_convert_type(v, jnp.int32)
    y = lax.bitcast_convert_type(MAGIC - (ib >> 1), jnp.float32)
    half = v * 0.5
    for _ in range(3):
        y = y * (1.5 - half * y * y)
    return jnp.where(v > 0.5, y, 0.0)


def _bf16r(v):
    ib = lax.bitcast_convert_type(v, jnp.int32)
    r = ib + 0x7FFF + ((ib >> 16) & 1)
    r = jnp.bitwise_and(r, jnp.int32(-65536))
    return lax.bitcast_convert_type(r, jnp.float32)


def _body(ei, xp, dfi, wg, bg, wp, bp, wv, bv, wt, tm,
          pol_out, val_out,
          d_s, p_s, s_s, stage_s,
          s_priv, ploc,
          rawc, rawr,
          xv, dvv, pvv,
          cb0, cb1, cb2, cb3,
          dbuf, pbuf, vbuf, shbuf):
    cid = lax.axis_index("c")
    tid = lax.axis_index("s")
    bl = tid // TPB
    b = cid * BPC + bl
    lane8 = tid % TPB
    iot = lax.iota(jnp.int32, L)
    ones16 = jnp.ones((L,), jnp.float32)
    row_base = (b * 2 + 0) * E_PAD
    col_base = (b * 2 + 1) * E_PAD
    start = lane8 * EPT
    nbase = tid * Q

    def zero_priv():
        def z16(i, _):
            s_priv[pl.ds(i * L, L)] = jnp.zeros((L,), jnp.float32)
            return 0
        lax.fori_loop(0, NLOC // L, z16, 0)

    def stage_and_reduce(acc):
        pltpu.sync_copy(s_priv, stage_s.at[pl.ds(tid * NLOC, NLOC)])
        plsc.subcore_barrier()
        rbase = bl * TPB * NLOC + lane8 * Q
        pltpu.sync_copy(stage_s.at[pl.ds(rbase, Q)], acc)
        for r in range(1, TPB):
            pltpu.sync_copy(stage_s.at[pl.ds(rbase + r * NLOC, Q)], xv)

            def radd(i, _):
                for u in range(4):
                    j = (i * 4 + u) * L
                    acc[pl.ds(j, L)] = acc[pl.ds(j, L)] + xv[pl.ds(j, L)]
                return 0
            lax.fori_loop(0, Q // L // 4, radd, 0)

    zero_priv()

    def deg_chunk(k, _):
        off = start + k * CH
        pltpu.sync_copy(ei.at[pl.ds(col_base + off, CH)], rawc)

        def cadd(i, _):
            for u in range(4):
                plsc.addupdate_scatter(
                    s_priv, [rawc[pl.ds((i * 4 + u) * L, L)]], ones16)
            return 0
        lax.fori_loop(0, CH // L // 4, cadd, 0)
        return 0
    lax.fori_loop(0, EPT // CH, deg_chunk, 0)
    stage_and_reduce(dvv)

    @pl.when(jnp.logical_and(cid == 0, tid == 0))
    def _selfloop_deg():
        dvv[pl.ds(0, L)] = dvv[pl.ds(0, L)] + jnp.where(iot == 0, 1.0, 0.0)

    pltpu.sync_copy(xp.at[pl.ds(cid * NPAD2 + nbase, Q)], xv)

    def dp16(i, _):
        dd = _rsqrt16(dvv[pl.ds(i * L, L)])
        dvv[pl.ds(i * L, L)] = dd
        pvv[pl.ds(i * L, L)] = dd * xv[pl.ds(i * L, L)]
        return 0
    lax.fori_loop(0, Q // L, dp16, 0)
    pltpu.sync_copy(dvv, d_s.at[pl.ds(nbase, Q)])
    pltpu.sync_copy(pvv, p_s.at[pl.ds(nbase, Q)])
    vbuf[...] = jnp.where(iot == 0, pvv[pl.ds(0, L)], 0.0)
    zero_priv()
    plsc.subcore_barrier()

    pltpu.sync_copy(p_s.at[pl.ds(bl * NLOC, NLOC)], ploc)

    def msg_chunk(k, _):
        off = start + k * CH
        pltpu.sync_copy(ei.at[pl.ds(row_base + off, CH)], rawr)
        pltpu.sync_copy(ei.at[pl.ds(col_base + off, CH)], rawc)

        def gadd(i, _):
            for u in range(4):
                pd16 = plsc.load_gather(
                    ploc, [rawr[pl.ds((i * 4 + u) * L, L)]])
                plsc.addupdate_scatter(
                    s_priv, [rawc[pl.ds((i * 4 + u) * L, L)]], pd16)
            return 0
        lax.fori_loop(0, CH // L // 4, gadd, 0)
        return 0
    lax.fori_loop(0, EPT // CH, msg_chunk, 0)

    @pl.when(jnp.logical_and(cid == 0, tid == 0))
    def _selfloop_msg():
        s_priv[pl.ds(0, L)] = s_priv[pl.ds(0, L)] + vbuf[...]

    stage_and_reduce(pvv)
    pltpu.sync_copy(d_s.at[pl.ds(nbase, Q)], xv)

    def smul(i, _):
        pvv[pl.ds(i * L, L)] = pvv[pl.ds(i * L, L)] * xv[pl.ds(i * L, L)]
        return 0
    lax.fori_loop(0, Q // L, smul, 0)
    pltpu.sync_copy(pvv, s_s.at[pl.ds(nbase, Q)])
    plsc.subcore_barrier()

    def lanesum(v):
        for sh in (8, 4, 2, 1):
            shbuf[...] = v
            v = v + plsc.load_gather(shbuf, [jnp.bitwise_and(iot + sh, L - 1)])
        return v

    @pl.when(tid < 2)
    def _value_head():
        base = tid * NLOC
        acc0 = jnp.zeros((L,), jnp.float32)

        def vchunk(k, acc):
            pltpu.sync_copy(s_s.at[pl.ds(base + k * Q, Q)], xv)

            def vacc(i, a):
                return a + xv[pl.ds(i * L, L)]
            return lax.fori_loop(0, Q // L, vacc, acc)
        acc0 = lax.fori_loop(0, N // Q, vchunk, acc0)
        tail = N - (N // Q) * Q
        nfull = tail // L
        rem = tail - nfull * L
        pltpu.sync_copy(s_s.at[pl.ds(base + (N // Q) * Q, tail)],
                        xv.at[pl.ds(0, tail)])

        def vacc2(i, a):
            return a + xv[pl.ds(i * L, L)]
        acc0 = lax.fori_loop(0, nfull, vacc2, acc0)
        acc0 = acc0 + jnp.where(iot < rem, xv[pl.ds(nfull * L, L)], 0.0)
        pltpu.sync_copy(wv, cb0)
        pltpu.sync_copy(wg, cb1)
        pltpu.sync_copy(bg, cb2)
        pltpu.sync_copy(bv, cb3)
        means = lanesum(acc0) / jnp.float32(N)
        emb16 = _bf16r(means * cb1[...] + cb2[...])
        valv = lanesum(emb16 * _bf16r(cb0[...])) + lanesum(cb3[...])
        vbuf[...] = jnp.where(iot == 0, valv, 0.0)
        pltpu.sync_copy(vbuf, val_out.at[pl.ds((cid * BPC + tid) * L, L)])

    @pl.when(jnp.logical_and(tid >= 2, tid < 4))
    def _policy_head():
        pb = tid - 2
        gb = cid * BPC + pb
        pltpu.sync_copy(dfi.at[pl.ds(gb * K, K)], dbuf)
        pltpu.sync_copy(s_s.at[pl.ds(pb * NLOC, NLOC)], ploc)
        pltpu.sync_copy(wp, cb0)
        pltpu.sync_copy(wg, cb1)
        pltpu.sync_copy(bg, cb2)
        pltpu.sync_copy(bp, cb3)
        bp_s = lanesum(cb3[...])
        wgv, bgv, wpv = cb1[...], cb2[...], cb0[...]
        wgc = [lanesum(jnp.where(iot == c, wgv, 0.0)) for c in range(8)]
        bgc = [lanesum(jnp.where(iot == c, bgv, 0.0)) for c in range(8)]
        wpc = [_bf16r(lanesum(jnp.where(iot == c, wpv, 0.0))) for c in range(8)]
        for j in range(K // L):
            sg16 = plsc.load_gather(ploc, [dbuf[pl.ds(j * L, L)]])
            acc = bp_s
            for c in range(8):
                acc = acc + _bf16r(sg16 * wgc[c] + bgc[c]) * wpc[c]
            pbuf[pl.ds(j * L, L)] = acc
        pltpu.sync_copy(wt, cb1)
        pltpu.sync_copy(tm, cb2)
        wait_l = lanesum(_bf16r(cb1[...]) * _bf16r(cb0[...])) + bp_s
        term_l = lanesum(_bf16r(cb2[...]) * _bf16r(cb0[...])) + bp_s
        pbuf[pl.ds(K, L)] = (jnp.where(iot == 0, wait_l, 0.0)
                             + jnp.where(iot == 1, term_l, 0.0))
        pltpu.sync_copy(pbuf, pol_out.at[pl.ds(gb * (K + L), K + L)])


@jax.jit
def _run(ei_pad, xp, dfi, wg, bg, wp, bp, wv, bv, wt, tm):
    f32, i32 = jnp.float32, jnp.int32
    kfn = pl.kernel(
        _body,
        out_type=[
            jax.ShapeDtypeStruct((B * (K + L),), f32),
            jax.ShapeDtypeStruct((B * L,), f32),
        ],
        mesh=plsc.VectorSubcoreMesh(core_axis_name="c", subcore_axis_name="s"),
        compiler_params=pltpu.CompilerParams(needs_layout_passes=False),
        scratch_types=[
            pltpu.VMEM_SHARED((NPAD2,), f32),
            pltpu.VMEM_SHARED((NPAD2,), f32),
            pltpu.VMEM_SHARED((NPAD2,), f32),
            pltpu.VMEM_SHARED((NS * NLOC,), f32),
            pltpu.VMEM((NLOC,), f32),
            pltpu.VMEM((NLOC,), f32),
            pltpu.VMEM((CH,), i32),
            pltpu.VMEM((CH,), i32),
            pltpu.VMEM((Q,), f32),
            pltpu.VMEM((Q,), f32),
            pltpu.VMEM((Q,), f32),
            pltpu.VMEM((L,), f32),
            pltpu.VMEM((L,), f32),
            pltpu.VMEM((L,), f32),
            pltpu.VMEM((L,), f32),
            pltpu.VMEM((K,), i32),
            pltpu.VMEM((K + L,), f32),
            pltpu.VMEM((L,), f32),
            pltpu.VMEM((L,), f32),
        ],
    )
    return kfn(ei_pad, xp, dfi, wg, bg, wp, bp, wv, bv, wt, tm)


def kernel(x, edge_index, defense_indices, W_gcn, b_gcn, wait_emb, term_emb,
           W_pol, b_pol, W_val, b_val):
    def pad16(a):
        a = a.reshape(-1).astype(jnp.float32)
        return jnp.pad(a, (0, L - a.shape[0]))

    wg, bg = pad16(W_gcn), pad16(b_gcn)
    wp, bp = pad16(W_pol), pad16(b_pol)
    wv, bv = pad16(W_val), pad16(b_val)
    wt, tm = pad16(wait_emb), pad16(term_emb)

    xp = jnp.pad(x.reshape(B, N), ((0, 0), (0, NLOC - N))).reshape(-1)
    pad_blk = jnp.full((B, 2, E_PAD - E), N, dtype=jnp.int32)
    ei_pad = jnp.concatenate([edge_index, pad_blk], axis=2).reshape(-1)

    pol_pad, val_pad = _run(ei_pad, xp, defense_indices.reshape(-1), wg, bg,
                            wp, bp, wv, bv, wt, tm)
    return pol_pad.reshape(B, K + L)[:, : K + 2], val_pad.reshape(B, L)[:, 0]
